# baseline (speedup 1.0000x reference)
import functools

import jax
import jax.numpy as jnp
from jax import lax
from jax.experimental import pallas as pl
from jax.experimental.pallas import tpu as pltpu
from jax.experimental.pallas import tpu_sc as plsc

B, C, H, W = 16, 768, 48, 48
K = 192
CBLK = 128
BBLK = 1
NW = 32
ROWS_PER_W = B * K // NW
CHUNK = 24
NCHUNK = ROWS_PER_W // CHUNK


def _select_body(x_ref, src_ref, e_ref):
    j = pl.program_id(0)
    b = pl.program_id(1)
    xb = x_ref[...]
    s = jnp.sum(xb * xb, axis=3)
    s = jnp.sum(s, axis=2)
    s = jnp.sum(s, axis=0)

    @pl.when(b == 0)
    def _():
        e_ref[0, pl.ds(j * CBLK, CBLK)] = s

    @pl.when(b > 0)
    def _():
        e_ref[0, pl.ds(j * CBLK, CBLK)] = e_ref[0, pl.ds(j * CBLK, CBLK)] + s

    @pl.when((j == (C // CBLK) - 1) & (b == (B // BBLK) - 1))
    def _():
        erow = e_ref[...]
        row_i = lax.broadcasted_iota(jnp.int32, (C, C), 0)
        col_i = lax.broadcasted_iota(jnp.int32, (C, C), 1)
        eye = (row_i == col_i).astype(jnp.float32)

        def split3(v):
            hi = v.astype(jnp.bfloat16).astype(jnp.float32)
            r1 = v - hi
            mid = r1.astype(jnp.bfloat16).astype(jnp.float32)
            return hi, mid, r1 - mid

        def transpose_row(v):
            out = jnp.zeros((C, 1), jnp.float32)
            for p in split3(v):
                out = out + lax.dot_general(eye, p, (((1,), (1,)), ((), ())),
                                            preferred_element_type=jnp.float32)
            return out

        ecol = transpose_row(erow)
        gt = ecol > erow
        eq = ecol == erow
        jlt = row_i < col_i
        m = jnp.where(gt | (eq & jlt), 1.0, 0.0)
        ones = jnp.ones((1, C), jnp.float32)
        rank = lax.dot_general(ones, m, (((1,), (0,)), ((), ())),
                               preferred_element_type=jnp.float32)
        rank_col = transpose_row(rank)
        slot = lax.broadcasted_iota(jnp.int32, (C, K), 1).astype(jnp.float32)
        onehot = jnp.where(rank_col == slot, 1.0, 0.0)
        cids = lax.broadcasted_iota(jnp.int32, (1, C), 1).astype(jnp.float32)
        inv = jnp.zeros((1, K), jnp.float32)
        for p in split3(cids):
            inv = inv + lax.dot_general(p, onehot, (((1,), (0,)), ((), ())),
                                        preferred_element_type=jnp.float32)
        src_ref[...] = inv.astype(jnp.int32)


_select = pl.pallas_call(
    _select_body,
    grid=(C // CBLK, B // BBLK),
    in_specs=[pl.BlockSpec((BBLK, CBLK, H, W), lambda j, b: (b, j, 0, 0))],
    out_specs=pl.BlockSpec((1, K), lambda j, b: (0, 0)),
    out_shape=jax.ShapeDtypeStruct((1, K), jnp.int32),
    scratch_shapes=[pltpu.VMEM((1, C), jnp.float32)],
)


NJ = K // NW


def _sc_gather_body(x_hbm, inv_hbm, out_hbm, inv_v, buf0, buf1, sem0, sem1):
    wid = lax.axis_index("s") * 2 + lax.axis_index("c")
    pltpu.sync_copy(inv_hbm, inv_v.at[pl.ds(0, K)])
    cvec = inv_v[pl.ds(NJ * wid, 16)]
    bufs = (buf0, buf1)
    sems = (sem0, sem1)
    copies = []
    NS = 2 * NJ
    for s in range(NS):
        t, hb = divmod(s, 2)
        c = cvec[t]
        copies.append(pltpu.async_copy(
            x_hbm.at[pl.ds(hb * (B // 2), B // 2), c], bufs[s % 2], sems[s % 2]))
        if s > 0:
            tp, hp = divmod(s - 1, 2)
            copies[s - 1].wait()
            pltpu.sync_copy(bufs[(s - 1) % 2],
                            out_hbm.at[pl.ds(hp * (B // 2), B // 2),
                                       NJ * wid + tp])
    tp, hp = divmod(NS - 1, 2)
    copies[NS - 1].wait()
    pltpu.sync_copy(bufs[(NS - 1) % 2],
                    out_hbm.at[pl.ds(hp * (B // 2), B // 2), NJ * wid + tp])


@functools.cache
def _sc_gather():
    mesh = plsc.VectorSubcoreMesh(core_axis_name="c", subcore_axis_name="s")
    return pl.kernel(
        _sc_gather_body,
        out_type=jax.ShapeDtypeStruct((B, K, H, W), jnp.float32),
        mesh=mesh,
        scratch_types=[
            pltpu.VMEM((K + 16,), jnp.int32),
            pltpu.VMEM((B // 2, H, W), jnp.float32),
            pltpu.VMEM((B // 2, H, W), jnp.float32),
            pltpu.SemaphoreType.DMA,
            pltpu.SemaphoreType.DMA,
        ],
    )


def kernel(x):
    inv = _select(x)
    return _sc_gather()(x, inv.reshape(K))


# speedup vs baseline: 1.1144x; 1.1144x over previous
import functools

import jax
import jax.numpy as jnp
from jax import lax
from jax.experimental import pallas as pl
from jax.experimental.pallas import tpu as pltpu
from jax.experimental.pallas import tpu_sc as plsc

B, C, H, W = 16, 768, 48, 48
K = 192
CBLK = 128
BBLK = 2
NW = 32
ROWS_PER_W = B * K // NW
CHUNK = 24
NCHUNK = ROWS_PER_W // CHUNK


def _select_body(x_ref, src_ref, e_ref):
    j = pl.program_id(0)
    b = pl.program_id(1)
    xb = x_ref[...]
    s = jnp.sum(xb * xb, axis=(0, 2, 3))

    @pl.when(b == 0)
    def _():
        e_ref[0, pl.ds(j * CBLK, CBLK)] = s

    @pl.when(b > 0)
    def _():
        e_ref[0, pl.ds(j * CBLK, CBLK)] = e_ref[0, pl.ds(j * CBLK, CBLK)] + s

    @pl.when((j == (C // CBLK) - 1) & (b == (B // BBLK) - 1))
    def _():
        erow = e_ref[...]
        row_i = lax.broadcasted_iota(jnp.int32, (C, C), 0)
        col_i = lax.broadcasted_iota(jnp.int32, (C, C), 1)
        eye = (row_i == col_i).astype(jnp.float32)

        def split3(v):
            hi = v.astype(jnp.bfloat16).astype(jnp.float32)
            r1 = v - hi
            mid = r1.astype(jnp.bfloat16).astype(jnp.float32)
            return hi, mid, r1 - mid

        def transpose_row(v):
            out = jnp.zeros((C, 1), jnp.float32)
            for p in split3(v):
                out = out + lax.dot_general(eye, p, (((1,), (1,)), ((), ())),
                                            preferred_element_type=jnp.float32)
            return out

        ecol = transpose_row(erow)
        gt = ecol > erow
        eq = ecol == erow
        jlt = row_i < col_i
        m = jnp.where(gt | (eq & jlt), 1.0, 0.0)
        ones = jnp.ones((1, C), jnp.float32)
        rank = lax.dot_general(ones, m, (((1,), (0,)), ((), ())),
                               preferred_element_type=jnp.float32)
        rank_col = transpose_row(rank)
        slot = lax.broadcasted_iota(jnp.int32, (C, K), 1).astype(jnp.float32)
        onehot = jnp.where(rank_col == slot, 1.0, 0.0)
        cids = lax.broadcasted_iota(jnp.int32, (1, C), 1).astype(jnp.float32)
        inv = jnp.zeros((1, K), jnp.float32)
        for p in split3(cids):
            inv = inv + lax.dot_general(p, onehot, (((1,), (0,)), ((), ())),
                                        preferred_element_type=jnp.float32)
        src_ref[...] = inv.astype(jnp.int32)


_select = pl.pallas_call(
    _select_body,
    grid=(C // CBLK, B // BBLK),
    in_specs=[pl.BlockSpec((BBLK, CBLK, H, W), lambda j, b: (b, j, 0, 0))],
    out_specs=pl.BlockSpec((1, K), lambda j, b: (0, 0)),
    out_shape=jax.ShapeDtypeStruct((1, K), jnp.int32),
    scratch_shapes=[pltpu.VMEM((1, C), jnp.float32)],
)


NJ = K // NW


def _sc_gather_body(x_hbm, inv_hbm, out_hbm, inv_v, buf0, buf1, sem0, sem1):
    wid = lax.axis_index("s") * 2 + lax.axis_index("c")
    pltpu.sync_copy(inv_hbm, inv_v.at[pl.ds(0, K)])
    cvec = inv_v[pl.ds(NJ * wid, 16)]
    bufs = (buf0, buf1)
    sems = (sem0, sem1)
    copies = []
    NS = 2 * NJ
    for s in range(NS):
        t, hb = divmod(s, 2)
        c = cvec[t]
        copies.append(pltpu.async_copy(
            x_hbm.at[pl.ds(hb * (B // 2), B // 2), c], bufs[s % 2], sems[s % 2]))
        if s > 0:
            tp, hp = divmod(s - 1, 2)
            copies[s - 1].wait()
            pltpu.sync_copy(bufs[(s - 1) % 2],
                            out_hbm.at[pl.ds(hp * (B // 2), B // 2),
                                       NJ * wid + tp])
    tp, hp = divmod(NS - 1, 2)
    copies[NS - 1].wait()
    pltpu.sync_copy(bufs[(NS - 1) % 2],
                    out_hbm.at[pl.ds(hp * (B // 2), B // 2), NJ * wid + tp])


@functools.cache
def _sc_gather():
    mesh = plsc.VectorSubcoreMesh(core_axis_name="c", subcore_axis_name="s")
    return pl.kernel(
        _sc_gather_body,
        out_type=jax.ShapeDtypeStruct((B, K, H, W), jnp.float32),
        mesh=mesh,
        scratch_types=[
            pltpu.VMEM((K + 16,), jnp.int32),
            pltpu.VMEM((B // 2, H, W), jnp.float32),
            pltpu.VMEM((B // 2, H, W), jnp.float32),
            pltpu.SemaphoreType.DMA,
            pltpu.SemaphoreType.DMA,
        ],
    )


def kernel(x):
    inv = _select(x)
    return _sc_gather()(x, inv.reshape(K))
